# baseline (speedup 1.0000x reference)
import functools

import jax
import jax.numpy as jnp
from jax import lax
from jax.experimental import pallas as pl
from jax.experimental.pallas import tpu as pltpu

FB = 2


def _encode_kernel(x_ref, W1c_ref, W2c_ref, b1L_ref, b2L_ref, mask_ref,
                   out_ref, E):
    for f in range(FB):
        xf = x_ref[f].astype(jnp.bfloat16)
        xstack = jnp.concatenate(
            [xf * mask_ref[pl.ds(e, 1)] for e in range(E)], axis=0
        )
        h = jnp.dot(W1c_ref[...], xstack,
                    preferred_element_type=jnp.float32) + b1L_ref[...]
        a = (h / (1.0 + jnp.abs(h))).astype(jnp.bfloat16)
        astack = jnp.concatenate(
            [a * mask_ref[pl.ds(e, 1)] for e in range(E)], axis=0
        )
        o = jnp.dot(W2c_ref[...], astack,
                    preferred_element_type=jnp.float32)
        out_ref[f] = o + b2L_ref[...]


@jax.jit
def kernel(x, Ws1, bs1, Ws2, bs2, eid):
    B, MAX_F, N = x.shape
    E, _, H = Ws1.shape
    P = Ws2.shape[-1]

    xt = jnp.transpose(x, (1, 2, 0))
    W1c = (jnp.transpose(Ws1, (2, 0, 1)).reshape(H, E * N)
           .astype(jnp.bfloat16))
    W2c = (jnp.transpose(Ws2, (2, 0, 1)).reshape(P, E * H)
           .astype(jnp.bfloat16))

    onehot = (eid[None, :] == jnp.arange(E, dtype=eid.dtype)[:, None])
    maskf = onehot.astype(jnp.float32)
    maskb = onehot.astype(jnp.bfloat16)
    b1L = jnp.matmul(bs1.T, maskf)
    b2L = jnp.matmul(bs2.T, maskf)

    grid = MAX_F // FB
    outT = pl.pallas_call(
        functools.partial(_encode_kernel, E=E),
        grid=(grid,),
        in_specs=[
            pl.BlockSpec((FB, N, B), lambda i: (i, 0, 0)),
            pl.BlockSpec((H, E * N), lambda i: (0, 0)),
            pl.BlockSpec((P, E * H), lambda i: (0, 0)),
            pl.BlockSpec((H, B), lambda i: (0, 0)),
            pl.BlockSpec((P, B), lambda i: (0, 0)),
            pl.BlockSpec((E, B), lambda i: (0, 0)),
        ],
        out_specs=pl.BlockSpec((FB, P, B), lambda i: (i, 0, 0)),
        out_shape=jax.ShapeDtypeStruct((MAX_F, P, B), jnp.float32),
    )(xt, W1c, W2c, b1L, b2L, maskb)

    return jnp.transpose(outT, (2, 0, 1))


# speedup vs baseline: 1.0263x; 1.0263x over previous
import functools

import jax
import jax.numpy as jnp
from jax import lax
from jax.experimental import pallas as pl
from jax.experimental.pallas import tpu as pltpu

FB = 5


def _encode_kernel(x_ref, W1c_ref, W2c_ref, b1L_ref, b2L_ref, mask_ref,
                   out_ref, E):
    for f in range(FB):
        xf = x_ref[f].astype(jnp.bfloat16)
        xstack = jnp.concatenate(
            [xf * mask_ref[pl.ds(e, 1)] for e in range(E)], axis=0
        )
        h = jnp.dot(W1c_ref[...], xstack,
                    preferred_element_type=jnp.float32) + b1L_ref[...]
        a = (h / (1.0 + jnp.abs(h))).astype(jnp.bfloat16)
        astack = jnp.concatenate(
            [a * mask_ref[pl.ds(e, 1)] for e in range(E)], axis=0
        )
        o = jnp.dot(W2c_ref[...], astack,
                    preferred_element_type=jnp.float32)
        out_ref[f] = o + b2L_ref[...]


@jax.jit
def kernel(x, Ws1, bs1, Ws2, bs2, eid):
    B, MAX_F, N = x.shape
    E, _, H = Ws1.shape
    P = Ws2.shape[-1]

    xt = jnp.transpose(x, (1, 2, 0))
    W1c = (jnp.transpose(Ws1, (2, 0, 1)).reshape(H, E * N)
           .astype(jnp.bfloat16))
    W2c = (jnp.transpose(Ws2, (2, 0, 1)).reshape(P, E * H)
           .astype(jnp.bfloat16))

    onehot = (eid[None, :] == jnp.arange(E, dtype=eid.dtype)[:, None])
    maskf = onehot.astype(jnp.float32)
    maskb = onehot.astype(jnp.bfloat16)
    b1L = jnp.matmul(bs1.T, maskf)
    b2L = jnp.matmul(bs2.T, maskf)

    grid = MAX_F // FB
    outT = pl.pallas_call(
        functools.partial(_encode_kernel, E=E),
        grid=(grid,),
        in_specs=[
            pl.BlockSpec((FB, N, B), lambda i: (i, 0, 0)),
            pl.BlockSpec((H, E * N), lambda i: (0, 0)),
            pl.BlockSpec((P, E * H), lambda i: (0, 0)),
            pl.BlockSpec((H, B), lambda i: (0, 0)),
            pl.BlockSpec((P, B), lambda i: (0, 0)),
            pl.BlockSpec((E, B), lambda i: (0, 0)),
        ],
        out_specs=pl.BlockSpec((FB, P, B), lambda i: (i, 0, 0)),
        out_shape=jax.ShapeDtypeStruct((MAX_F, P, B), jnp.float32),
    )(xt, W1c, W2c, b1L, b2L, maskb)

    return jnp.transpose(outT, (2, 0, 1))
